# baseline (speedup 1.0000x reference)
import functools

import jax
import jax.numpy as jnp
from jax import lax
from jax.experimental import pallas as pl
from jax.experimental.pallas import tpu as pltpu
from jax.experimental.pallas import tpu_sc as plsc

B, N, D, H, E = 32, 512, 16, 128, 64
S = B * N
NC, NS, L = 2, 16, 16
NW = NC * NS
SLOTS_W = S // NW
G = 8
ROWS = G * D
NG = SLOTS_W // G
HS = H // L


def _lane_bcast(vec, d):
    return lax.gather(
        vec, jnp.broadcast_to(d, (L,))[:, None],
        lax.GatherDimensionNumbers(
            offset_dims=(), collapsed_slice_dims=(0,), start_index_map=(0,)),
        slice_sizes=(1,),
        mode=lax.GatherScatterMode.PROMISE_IN_BOUNDS)


def _make_agg_kernel():
    mesh = plsc.VectorSubcoreMesh(core_axis_name="c", subcore_axis_name="s")

    @functools.partial(
        pl.kernel,
        out_type=jax.ShapeDtypeStruct((2, S, H), jnp.float32),
        name="neighbor_agg",
        mesh=mesh,
        scratch_types=[
            pltpu.VMEM((2, SLOTS_W * D), jnp.int32),
            pltpu.VMEM((2, SLOTS_W * D), jnp.float32),
            pltpu.VMEM((2, ROWS, H), jnp.float32),
            pltpu.VMEM((2, G, H), jnp.float32),
            pltpu.SemaphoreType.DMA,
            pltpu.SemaphoreType.DMA,
            pltpu.SemaphoreType.DMA,
            pltpu.SemaphoreType.DMA,
        ],
    )
    def agg_kernel(nh, idx2, w2, agg_out, idx_v, w_v, rows_v, out_v,
                   sr0, sr1, so0, so1):
        wid = lax.axis_index("s") * NC + lax.axis_index("c")
        base = wid * SLOTS_W
        srs = (sr0, sr1)
        sos = (so0, so1)

        for dirn in range(2):
            pltpu.sync_copy(idx2.at[dirn, pl.ds(base * D, SLOTS_W * D)],
                            idx_v.at[dirn])
            pltpu.sync_copy(w2.at[dirn, pl.ds(base * D, SLOTS_W * D)],
                            w_v.at[dirn])

        T = 2 * NG

        def fire(t, b):
            dirn = t // NG
            g = t - dirn * NG
            pltpu.async_copy(
                nh.at[idx_v.at[dirn, pl.ds(g * ROWS, ROWS)]],
                rows_v.at[b], srs[b])

        def consume(t, b):
            dirn = t // NG
            g = t - dirn * NG
            pltpu.make_async_copy(
                nh.at[idx_v.at[dirn, pl.ds(g * ROWS, ROWS)]],
                rows_v.at[b], srs[b]).wait()
            @pl.when(t >= 2)
            def _():
                pltpu.make_async_copy(
                    out_v.at[b], agg_out.at[0, pl.ds(base, G)],
                    sos[b]).wait()
            for s in range(G):
                wv16 = w_v[dirn, pl.ds(g * ROWS + s * D, L)]
                accs = [jnp.zeros((L,), jnp.float32) for _ in range(HS)]
                for d in range(D):
                    r = s * D + d
                    wb = _lane_bcast(wv16, d)
                    for h in range(HS):
                        accs[h] = accs[h] + wb * rows_v[b, r, pl.ds(h * L, L)]
                for h in range(HS):
                    out_v[b, s, pl.ds(h * L, L)] = accs[h]
            pltpu.async_copy(
                out_v.at[b], agg_out.at[dirn, pl.ds(base + g * G, G)],
                sos[b])

        fire(0, 0)

        @pl.loop(0, T // 2)
        def _pipe(i):
            t0 = 2 * i
            fire(t0 + 1, 1)
            consume(t0, 0)

            @pl.when(t0 + 2 < T)
            def _():
                fire(t0 + 2, 0)
            consume(t0 + 1, 1)

        for b in range(2):
            pltpu.make_async_copy(
                out_v.at[b], agg_out.at[0, pl.ds(base, G)],
                sos[b]).wait()

    return agg_kernel


_agg = _make_agg_kernel()

BLK = 512
NBLK = S // BLK


def _gates_body(inagg_ref, outagg_ref, labin_ref, win_ref, labout_ref,
                wout_ref, cell_ref, wni_ref, uti_ref, wno_ref, uto_ref,
                edge_ref, b_ref, out_ref):
    iota_e = lax.broadcasted_iota(jnp.int32, (BLK, E), 1)

    def hist(lab, w):
        acc = jnp.zeros((BLK, E), jnp.float32)
        for d in range(D):
            acc = acc + jnp.where(lab[:, d:d + 1] == iota_e, w[:, d:d + 1], 0.0)
        return acc

    hin = hist(labin_ref[...], win_ref[...])
    hout = hist(labout_ref[...], wout_ref[...])
    eui = jnp.dot(edge_ref[...], uti_ref[...], preferred_element_type=jnp.float32)
    euo = jnp.dot(edge_ref[...], uto_ref[...], preferred_element_type=jnp.float32)
    pre = (jnp.dot(inagg_ref[...], wni_ref[...], preferred_element_type=jnp.float32)
           + jnp.dot(hin, eui, preferred_element_type=jnp.float32)
           + jnp.dot(outagg_ref[...], wno_ref[...], preferred_element_type=jnp.float32)
           + jnp.dot(hout, euo, preferred_element_type=jnp.float32)
           + b_ref[...])
    i_g = jax.nn.sigmoid(pre[:, 0:H])
    o_g = jax.nn.sigmoid(pre[:, H:2 * H])
    f_g = jax.nn.sigmoid(pre[:, 2 * H:3 * H])
    c_t = jnp.tanh(pre[:, 3 * H:4 * H])
    new_cell = f_g * cell_ref[...] + i_g * c_t
    out_ref[...] = o_g * jnp.tanh(new_cell)


def _row_spec(cols):
    return pl.BlockSpec((BLK, cols), lambda i: (i, 0))


def _full_spec(shape):
    return pl.BlockSpec(shape, lambda i: (0,) * len(shape))


_gates = pl.pallas_call(
    _gates_body,
    grid=(NBLK,),
    in_specs=[
        _row_spec(H), _row_spec(H),
        _row_spec(D), _row_spec(D), _row_spec(D), _row_spec(D),
        _row_spec(H),
        _full_spec((H, 4 * H)), _full_spec((H, 4 * H)),
        _full_spec((H, 4 * H)), _full_spec((H, 4 * H)),
        _full_spec((E, H)), _full_spec((1, 4 * H)),
    ],
    out_specs=pl.BlockSpec((BLK, H), lambda i: (i, 0)),
    out_shape=jax.ShapeDtypeStruct((S, H), jnp.float32),
)


def kernel(node_hidden, cell, in_node_mask, out_node_mask, W_in, U_in,
           W_out, U_out, b, edge_emb, in_nodes, in_labels, out_nodes,
           out_labels):
    nh = node_hidden.reshape(S, H)
    boff = (jnp.arange(B, dtype=jnp.int32) * N)[:, None, None]
    idx2 = jnp.stack([(in_nodes + boff).reshape(S * D),
                      (out_nodes + boff).reshape(S * D)])
    w2 = jnp.stack([in_node_mask.reshape(S * D),
                    out_node_mask.reshape(S * D)])
    agg = _agg(nh, idx2, w2)

    wni = W_in.transpose(1, 0, 2).reshape(H, 4 * H)
    uti = U_in.transpose(1, 0, 2).reshape(H, 4 * H)
    wno = W_out.transpose(1, 0, 2).reshape(H, 4 * H)
    uto = U_out.transpose(1, 0, 2).reshape(H, 4 * H)
    b_flat = b.reshape(1, 4 * H)

    new_h = _gates(agg[0], agg[1],
                   in_labels.reshape(S, D), in_node_mask.reshape(S, D),
                   out_labels.reshape(S, D), out_node_mask.reshape(S, D),
                   cell.reshape(S, H),
                   wni, uti, wno, uto, edge_emb, b_flat)
    return new_h.reshape(B, N, H)


# speedup vs baseline: 1.7582x; 1.7582x over previous
import functools

import jax
import jax.numpy as jnp
from jax import lax
from jax.experimental import pallas as pl
from jax.experimental.pallas import tpu as pltpu
from jax.experimental.pallas import tpu_sc as plsc

B, N, D, H, E = 32, 512, 16, 128, 64
S = B * N
NC, NS, L = 2, 16, 16
NW = NC * NS
SLOTS_W = S // NW
G = 8
ROWS = G * D
NG = SLOTS_W // G
HS = H // L


def _lane_bcast(vec, d):
    return lax.gather(
        vec, jnp.broadcast_to(d, (L,))[:, None],
        lax.GatherDimensionNumbers(
            offset_dims=(), collapsed_slice_dims=(0,), start_index_map=(0,)),
        slice_sizes=(1,),
        mode=lax.GatherScatterMode.PROMISE_IN_BOUNDS)


def _make_agg_kernel():
    mesh = plsc.VectorSubcoreMesh(core_axis_name="c", subcore_axis_name="s")

    @functools.partial(
        pl.kernel,
        out_type=jax.ShapeDtypeStruct((2, S, H), jnp.float32),
        name="neighbor_agg",
        mesh=mesh,
        scratch_types=[
            pltpu.VMEM((2, SLOTS_W * D), jnp.int32),
            pltpu.VMEM((2, SLOTS_W * D), jnp.float32),
            pltpu.VMEM((2, ROWS, H), jnp.float32),
            pltpu.VMEM((2, G, H), jnp.float32),
            pltpu.SemaphoreType.DMA,
            pltpu.SemaphoreType.DMA,
            pltpu.SemaphoreType.DMA,
            pltpu.SemaphoreType.DMA,
        ],
    )
    def agg_kernel(nh, idx2, w2, agg_out, idx_v, w_v, rows_v, out_v,
                   sr0, sr1, so0, so1):
        wid = lax.axis_index("s") * NC + lax.axis_index("c")
        base = wid * SLOTS_W
        srs = (sr0, sr1)
        sos = (so0, so1)

        for dirn in range(2):
            pltpu.sync_copy(idx2.at[dirn, pl.ds(base * D, SLOTS_W * D)],
                            idx_v.at[dirn])
            pltpu.sync_copy(w2.at[dirn, pl.ds(base * D, SLOTS_W * D)],
                            w_v.at[dirn])

        for dirn in range(2):
            def fire(g, b, dirn=dirn):
                pltpu.async_copy(
                    nh.at[idx_v.at[dirn, pl.ds(g * ROWS, ROWS)]],
                    rows_v.at[b], srs[b])

            def consume(g, b, dirn=dirn):
                pltpu.make_async_copy(
                    nh.at[idx_v.at[dirn, pl.ds(g * ROWS, ROWS)]],
                    rows_v.at[b], srs[b]).wait()
                @pl.when(g >= 2)
                def _():
                    pltpu.make_async_copy(
                        out_v.at[b], agg_out.at[dirn, pl.ds(base, G)],
                        sos[b]).wait()
                for s in range(G):
                    wv16 = w_v[dirn, pl.ds(g * ROWS + s * D, L)]

                    def dbody(d, accs, s=s, wv16=wv16):
                        r = s * D + d
                        wb = _lane_bcast(wv16, d)
                        return tuple(
                            accs[h] + wb * rows_v[b, r, pl.ds(h * L, L)]
                            for h in range(HS)
                        )
                    accs = lax.fori_loop(
                        0, D, dbody,
                        tuple(jnp.zeros((L,), jnp.float32) for _ in range(HS)))
                    for h in range(HS):
                        out_v[b, s, pl.ds(h * L, L)] = accs[h]
                pltpu.async_copy(
                    out_v.at[b], agg_out.at[dirn, pl.ds(base + g * G, G)],
                    sos[b])

            fire(0, 0)

            @pl.loop(0, NG // 2)
            def _pipe(i, dirn=dirn):
                t0 = 2 * i
                fire(t0 + 1, 1)
                consume(t0, 0)

                @pl.when(t0 + 2 < NG)
                def _():
                    fire(t0 + 2, 0)
                consume(t0 + 1, 1)

            for b in range(2):
                pltpu.make_async_copy(
                    out_v.at[b], agg_out.at[dirn, pl.ds(base, G)],
                    sos[b]).wait()

    return agg_kernel


_agg = _make_agg_kernel()

BLK = 512
NBLK = S // BLK


def _hist_body(labin_ref, win_ref, labout_ref, wout_ref, hin_ref, hout_ref):
    lin = labin_ref[...]
    win = win_ref[...]
    lout = labout_ref[...]
    wout = wout_ref[...]
    for e in range(E):
        hin_ref[e, :] = jnp.sum(jnp.where(lin == e, win, 0.0), axis=0)
        hout_ref[e, :] = jnp.sum(jnp.where(lout == e, wout, 0.0), axis=0)


def _lab_spec():
    return pl.BlockSpec((D, BLK), lambda i: (0, i))


_hist = pl.pallas_call(
    _hist_body,
    grid=(NBLK,),
    in_specs=[_lab_spec(), _lab_spec(), _lab_spec(), _lab_spec()],
    out_specs=[pl.BlockSpec((E, BLK), lambda i: (0, i)),
               pl.BlockSpec((E, BLK), lambda i: (0, i))],
    out_shape=[jax.ShapeDtypeStruct((E, S), jnp.float32),
               jax.ShapeDtypeStruct((E, S), jnp.float32)],
)


def _gates_body(inagg_ref, outagg_ref, hin_ref, hout_ref, cell_ref,
                wni_ref, uti_ref, wno_ref, uto_ref, edge_ref, b_ref,
                out_ref):
    eui = jnp.dot(edge_ref[...], uti_ref[...], preferred_element_type=jnp.float32)
    euo = jnp.dot(edge_ref[...], uto_ref[...], preferred_element_type=jnp.float32)
    dn = (((0,), (0,)), ((), ()))
    pre = (jnp.dot(inagg_ref[...], wni_ref[...], preferred_element_type=jnp.float32)
           + lax.dot_general(hin_ref[...], eui, dn, preferred_element_type=jnp.float32)
           + jnp.dot(outagg_ref[...], wno_ref[...], preferred_element_type=jnp.float32)
           + lax.dot_general(hout_ref[...], euo, dn, preferred_element_type=jnp.float32)
           + b_ref[...])
    i_g = jax.nn.sigmoid(pre[:, 0:H])
    o_g = jax.nn.sigmoid(pre[:, H:2 * H])
    f_g = jax.nn.sigmoid(pre[:, 2 * H:3 * H])
    c_t = jnp.tanh(pre[:, 3 * H:4 * H])
    new_cell = f_g * cell_ref[...] + i_g * c_t
    out_ref[...] = o_g * jnp.tanh(new_cell)


def _row_spec(cols):
    return pl.BlockSpec((BLK, cols), lambda i: (i, 0))


def _full_spec(shape):
    return pl.BlockSpec(shape, lambda i: (0,) * len(shape))


_gates = pl.pallas_call(
    _gates_body,
    grid=(NBLK,),
    in_specs=[
        _row_spec(H), _row_spec(H),
        pl.BlockSpec((E, BLK), lambda i: (0, i)),
        pl.BlockSpec((E, BLK), lambda i: (0, i)),
        _row_spec(H),
        _full_spec((H, 4 * H)), _full_spec((H, 4 * H)),
        _full_spec((H, 4 * H)), _full_spec((H, 4 * H)),
        _full_spec((E, H)), _full_spec((1, 4 * H)),
    ],
    out_specs=pl.BlockSpec((BLK, H), lambda i: (i, 0)),
    out_shape=jax.ShapeDtypeStruct((S, H), jnp.float32),
)


def kernel(node_hidden, cell, in_node_mask, out_node_mask, W_in, U_in,
           W_out, U_out, b, edge_emb, in_nodes, in_labels, out_nodes,
           out_labels):
    nh = node_hidden.reshape(S, H)
    boff = (jnp.arange(B, dtype=jnp.int32) * N)[:, None, None]
    idx2 = jnp.stack([(in_nodes + boff).reshape(S * D),
                      (out_nodes + boff).reshape(S * D)])
    w2 = jnp.stack([in_node_mask.reshape(S * D),
                    out_node_mask.reshape(S * D)])
    hin, hout = _hist(in_labels.reshape(S, D).T,
                      in_node_mask.reshape(S, D).T,
                      out_labels.reshape(S, D).T,
                      out_node_mask.reshape(S, D).T)
    agg = _agg(nh, idx2, w2)

    wni = W_in.transpose(1, 0, 2).reshape(H, 4 * H)
    uti = U_in.transpose(1, 0, 2).reshape(H, 4 * H)
    wno = W_out.transpose(1, 0, 2).reshape(H, 4 * H)
    uto = U_out.transpose(1, 0, 2).reshape(H, 4 * H)
    b_flat = b.reshape(1, 4 * H)

    new_h = _gates(agg[0], agg[1], hin, hout, cell.reshape(S, H),
                   wni, uti, wno, uto, edge_emb, b_flat)
    return new_h.reshape(B, N, H)
